# baseline (device time: 28998 ns/iter reference)
import jax
import jax.numpy as jnp
from jax import lax
from jax.experimental import pallas as pl
from jax.experimental.pallas import tpu as pltpu

N_DEV = 32
N_STEPS = 5
BLK = 64

BITS_A = (0, 1, 2, 3, 4)
BITS_B = (3, 4, 0, 1, 2)


def kernel(x, Wq, K_ext, V_ext, Wo):
    B, Sq, Dm = x.shape
    _, Skv_loc, Hq, Dh = K_ext.shape
    C_COLS = Sq * Dh
    W_COLS = C_COLS + Sq

    def body(x_ref, wq_ref, k_ref, v_ref, wo_ref, out_ref,
             commA, commB, recvA, recvB, ctx_buf,
             sendA_sems, recvA_sems, sendB_sems, recvB_sems):
        my = lax.axis_index("i")

        barrier_sem = pltpu.get_barrier_semaphore()
        for k in range(N_STEPS):
            pl.semaphore_signal(
                barrier_sem, inc=1,
                device_id=(my ^ (1 << k),),
                device_id_type=pl.DeviceIdType.MESH,
            )

        xf = x_ref[...].reshape(B * Sq, Dm)
        q = jnp.dot(xf, wq_ref[...], preferred_element_type=jnp.float32)
        q = q.reshape(B, Sq, Hq, Dh)

        qb = lax.broadcasted_iota(jnp.int32, (Sq, Skv_loc), 0) // BLK
        kb = my * (Skv_loc // BLK) + (
            lax.broadcasted_iota(jnp.int32, (Sq, Skv_loc), 1) // BLK
        )
        mask = (qb == kb) | (kb == 0) | ((qb + kb) % 3 == 0)

        k_all = k_ref[...]
        v_all = v_ref[...]

        def partials(b, comm):
            s = jnp.einsum(
                "ihd,jhd->hij", q[b], k_all[b],
                preferred_element_type=jnp.float32,
            ) * 0.125
            w = jnp.where(mask[None], jnp.exp(s), 0.0)
            l = jnp.sum(w, axis=-1)
            c = jnp.einsum(
                "hij,jhd->hid", w, v_all[b],
                preferred_element_type=jnp.float32,
            )
            comm[0, :, :C_COLS] = c.reshape(Hq, C_COLS).astype(jnp.bfloat16)
            comm[0, :, C_COLS:] = l.astype(jnp.bfloat16)

        def make(chain_comm, chain_recv, ssems, rsems, bit, step):
            return pltpu.make_async_remote_copy(
                src_ref=chain_comm.at[step],
                dst_ref=chain_recv.at[step],
                send_sem=ssems.at[step],
                recv_sem=rsems.at[step],
                device_id=(my ^ (1 << bit),),
                device_id_type=pl.DeviceIdType.MESH,
            )

        partials(0, commA)
        pl.semaphore_wait(barrier_sem, N_STEPS)
        rA = make(commA, recvA, sendA_sems, recvA_sems, BITS_A[0], 0)
        rA.start()
        partials(1, commB)
        rB = make(commB, recvB, sendB_sems, recvB_sems, BITS_B[0], 0)
        rB.start()

        rdmas = [rA, rB]
        for step in range(N_STEPS):
            rA.wait_recv()
            commA[step + 1] = commA[step] + recvA[step]
            if step + 1 < N_STEPS:
                rA = make(commA, recvA, sendA_sems, recvA_sems,
                          BITS_A[step + 1], step + 1)
                rA.start()
                rdmas.append(rA)
            rB.wait_recv()
            commB[step + 1] = commB[step] + recvB[step]
            if step + 1 < N_STEPS:
                rB = make(commB, recvB, sendB_sems, recvB_sems,
                          BITS_B[step + 1], step + 1)
                rB.start()
                rdmas.append(rB)
        for r in rdmas:
            r.wait_send()

        for b, comm in ((0, commA), (1, commB)):
            c3 = comm[N_STEPS, :, :C_COLS].reshape(Hq, Sq, Dh)
            l2 = comm[N_STEPS, :, C_COLS:]
            for h in range(Hq):
                ctx_bh = c3[h].astype(jnp.float32) * (
                    1.0 / l2[h].astype(jnp.float32)[:, None]
                )
                ctx_buf[b * Sq:(b + 1) * Sq, h * Dh:(h + 1) * Dh] = ctx_bh
        out = jnp.dot(
            ctx_buf[...], wo_ref[...], preferred_element_type=jnp.float32
        )
        out_ref[...] = out.reshape(B, Sq, Dm)

    return pl.pallas_call(
        body,
        out_shape=jax.ShapeDtypeStruct((B, Sq, Dm), jnp.float32),
        in_specs=[pl.BlockSpec(memory_space=pltpu.VMEM)] * 5,
        out_specs=pl.BlockSpec(memory_space=pltpu.VMEM),
        scratch_shapes=[
            pltpu.VMEM((N_STEPS + 1, Hq, W_COLS), jnp.bfloat16),
            pltpu.VMEM((N_STEPS + 1, Hq, W_COLS), jnp.bfloat16),
            pltpu.VMEM((N_STEPS, Hq, W_COLS), jnp.bfloat16),
            pltpu.VMEM((N_STEPS, Hq, W_COLS), jnp.bfloat16),
            pltpu.VMEM((B * Sq, Hq * Dh), jnp.float32),
            pltpu.SemaphoreType.DMA((N_STEPS,)),
            pltpu.SemaphoreType.DMA((N_STEPS,)),
            pltpu.SemaphoreType.DMA((N_STEPS,)),
            pltpu.SemaphoreType.DMA((N_STEPS,)),
        ],
        compiler_params=pltpu.CompilerParams(collective_id=0),
    )(x, Wq, K_ext, V_ext, Wo)


# device time: 28005 ns/iter; 1.0355x vs baseline; 1.0355x over previous
import jax
import jax.numpy as jnp
from jax import lax
from jax.experimental import pallas as pl
from jax.experimental.pallas import tpu as pltpu

N_DEV = 32
N_STEPS = 5
BLK = 64

BITS_A = (0, 1, 2, 3, 4)
BITS_B = (3, 4, 0, 1, 2)


def kernel(x, Wq, K_ext, V_ext, Wo):
    B, Sq, Dm = x.shape
    _, Skv_loc, Hq, Dh = K_ext.shape
    HD = Hq * Dh
    W_COLS = HD + Hq

    def body(x_ref, wq_ref, k_ref, v_ref, wo_ref, out_ref,
             commA, commB, recvA, recvB, ctx_buf,
             sendA_sems, recvA_sems, sendB_sems, recvB_sems):
        my = lax.axis_index("i")

        barrier_sem = pltpu.get_barrier_semaphore()
        for k in range(N_STEPS):
            pl.semaphore_signal(
                barrier_sem, inc=1,
                device_id=(my ^ (1 << k),),
                device_id_type=pl.DeviceIdType.MESH,
            )

        xf = x_ref[...].reshape(B * Sq, Dm)
        q2 = jnp.dot(xf, wq_ref[...], preferred_element_type=jnp.float32)
        k2 = k_ref[...].reshape(B, Skv_loc, HD)
        v2 = v_ref[...].reshape(B, Skv_loc, HD)

        qb = lax.broadcasted_iota(jnp.int32, (Sq, Skv_loc), 0) // BLK
        kb = my * (Skv_loc // BLK) + (
            lax.broadcasted_iota(jnp.int32, (Sq, Skv_loc), 1) // BLK
        )
        mask = (qb == kb) | (kb == 0) | ((qb + kb) % 3 == 0)

        def partials(b, comm):
            q_b = q2[b * Sq:(b + 1) * Sq, :]
            k_b = k2[b]
            v_b = v2[b]
            for h in range(Hq):
                cols = slice(h * Dh, (h + 1) * Dh)
                s = lax.dot_general(
                    q_b[:, cols], k_b[:, cols],
                    (((1,), (1,)), ((), ())),
                    preferred_element_type=jnp.float32,
                ) * 0.125
                w = jnp.where(mask, jnp.exp(s), 0.0)
                l_h = jnp.sum(w, axis=1)
                c_h = jnp.dot(
                    w, v_b[:, cols], preferred_element_type=jnp.float32
                )
                comm[0, :, cols] = c_h.astype(jnp.bfloat16)
                comm[0, :, HD + h:HD + h + 1] = (
                    l_h[:, None].astype(jnp.bfloat16)
                )

        def make(chain_comm, chain_recv, ssems, rsems, bit, step):
            return pltpu.make_async_remote_copy(
                src_ref=chain_comm.at[step],
                dst_ref=chain_recv.at[step],
                send_sem=ssems.at[step],
                recv_sem=rsems.at[step],
                device_id=(my ^ (1 << bit),),
                device_id_type=pl.DeviceIdType.MESH,
            )

        def finalize(b, comm):
            tot = comm[N_STEPS]
            recip = 1.0 / tot[:, HD:HD + Hq].astype(jnp.float32)
            mult = jnp.broadcast_to(
                recip[:, :, None], (Sq, Hq, Dh)
            ).reshape(Sq, HD)
            ctx_buf[b * Sq:(b + 1) * Sq, :] = (
                tot[:, :HD].astype(jnp.float32) * mult
            )

        partials(0, commA)
        pl.semaphore_wait(barrier_sem, N_STEPS)
        rA = make(commA, recvA, sendA_sems, recvA_sems, BITS_A[0], 0)
        rA.start()
        partials(1, commB)
        rB = make(commB, recvB, sendB_sems, recvB_sems, BITS_B[0], 0)
        rB.start()

        rdmas = [rA, rB]
        for step in range(N_STEPS):
            rA.wait_recv()
            commA[step + 1] = commA[step] + recvA[step]
            if step + 1 < N_STEPS:
                rA = make(commA, recvA, sendA_sems, recvA_sems,
                          BITS_A[step + 1], step + 1)
                rA.start()
                rdmas.append(rA)
            if step + 1 == N_STEPS:
                finalize(0, commA)
            rB.wait_recv()
            commB[step + 1] = commB[step] + recvB[step]
            if step + 1 < N_STEPS:
                rB = make(commB, recvB, sendB_sems, recvB_sems,
                          BITS_B[step + 1], step + 1)
                rB.start()
                rdmas.append(rB)
        finalize(1, commB)
        for r in rdmas:
            r.wait_send()

        out = jnp.dot(
            ctx_buf[...], wo_ref[...], preferred_element_type=jnp.float32
        )
        out_ref[...] = out.reshape(B, Sq, Dm)

    return pl.pallas_call(
        body,
        out_shape=jax.ShapeDtypeStruct((B, Sq, Dm), jnp.float32),
        in_specs=[pl.BlockSpec(memory_space=pltpu.VMEM)] * 5,
        out_specs=pl.BlockSpec(memory_space=pltpu.VMEM),
        scratch_shapes=[
            pltpu.VMEM((N_STEPS + 1, Sq, W_COLS), jnp.bfloat16),
            pltpu.VMEM((N_STEPS + 1, Sq, W_COLS), jnp.bfloat16),
            pltpu.VMEM((N_STEPS, Sq, W_COLS), jnp.bfloat16),
            pltpu.VMEM((N_STEPS, Sq, W_COLS), jnp.bfloat16),
            pltpu.VMEM((B * Sq, HD), jnp.float32),
            pltpu.SemaphoreType.DMA((N_STEPS,)),
            pltpu.SemaphoreType.DMA((N_STEPS,)),
            pltpu.SemaphoreType.DMA((N_STEPS,)),
            pltpu.SemaphoreType.DMA((N_STEPS,)),
        ],
        compiler_params=pltpu.CompilerParams(collective_id=0),
    )(x, Wq, K_ext, V_ext, Wo)


# device time: 24571 ns/iter; 1.1802x vs baseline; 1.1398x over previous
import jax
import jax.numpy as jnp
from jax import lax
from jax.experimental import pallas as pl
from jax.experimental.pallas import tpu as pltpu

N_DEV = 32
N_PHASES = 3
N_SLOTS = 7
BLK = 64


def kernel(x, Wq, K_ext, V_ext, Wo):
    B, Sq, Dm = x.shape
    _, Skv_loc, Hq, Dh = K_ext.shape
    HD = Hq * Dh
    W_COLS = HD + Hq

    def body(x_ref, wq_ref, k_ref, v_ref, wo_ref, out_ref,
             commA, commB, recvA, recvB, ctx_buf,
             sendA_sems, recvA_sems, sendB_sems, recvB_sems):
        my = lax.axis_index("i")

        zz = my // 8
        p = my % 8
        yy = p // 2
        xx = (p % 2 + yy) % 2

        def snake(xv, yv):
            return 2 * yv + (xv + yv) % 2

        peers_x = [my ^ 1]
        peers_y = [8 * zz + snake(xx, (yy + d) % 4) for d in (1, 2, 3)]
        peers_z = [8 * ((zz + d) % 4) + p for d in (1, 2, 3)]

        PH_X = (peers_x, [0])
        PH_Y = (peers_y, [1, 2, 3])
        PH_Z = (peers_z, [4, 5, 6])
        phases_A = [PH_X, PH_Y, PH_Z]
        phases_B = [PH_Y, PH_Z, PH_X]

        barrier_sem = pltpu.get_barrier_semaphore()
        all_peers = peers_x + peers_y + peers_z
        for tgt in all_peers:
            pl.semaphore_signal(
                barrier_sem, inc=1,
                device_id=(tgt,),
                device_id_type=pl.DeviceIdType.MESH,
            )

        xf = x_ref[...].reshape(B * Sq, Dm)
        q2 = jnp.dot(xf, wq_ref[...], preferred_element_type=jnp.float32)
        k2 = k_ref[...].reshape(B, Skv_loc, HD)
        v2 = v_ref[...].reshape(B, Skv_loc, HD)

        qb = lax.broadcasted_iota(jnp.int32, (Sq, Skv_loc), 0) // BLK
        kb = my * (Skv_loc // BLK) + (
            lax.broadcasted_iota(jnp.int32, (Sq, Skv_loc), 1) // BLK
        )
        mask = (qb == kb) | (kb == 0) | ((qb + kb) % 3 == 0)

        def partials(b, comm):
            q_b = q2[b * Sq:(b + 1) * Sq, :]
            k_b = k2[b]
            v_b = v2[b]
            for h in range(Hq):
                cols = slice(h * Dh, (h + 1) * Dh)
                s = lax.dot_general(
                    q_b[:, cols], k_b[:, cols],
                    (((1,), (1,)), ((), ())),
                    preferred_element_type=jnp.float32,
                ) * 0.125
                w = jnp.where(mask, jnp.exp(s), 0.0)
                l_h = jnp.sum(w, axis=1)
                c_h = jnp.dot(
                    w, v_b[:, cols], preferred_element_type=jnp.float32
                )
                comm[0, :, cols] = c_h.astype(jnp.bfloat16)
                comm[0, :, HD + h:HD + h + 1] = (
                    l_h[:, None].astype(jnp.bfloat16)
                )

        def launch(comm, recv, ssems, rsems, phase_idx, phase):
            peers, slots = phase
            rs = []
            for tgt, sl in zip(peers, slots):
                r = pltpu.make_async_remote_copy(
                    src_ref=comm.at[phase_idx],
                    dst_ref=recv.at[sl],
                    send_sem=ssems.at[sl],
                    recv_sem=rsems.at[sl],
                    device_id=(tgt,),
                    device_id_type=pl.DeviceIdType.MESH,
                )
                r.start()
                rs.append(r)
            return rs

        def finish(comm, recv, phase_idx, phase, rs):
            _, slots = phase
            for r in rs:
                r.wait_recv()
            acc = comm[phase_idx]
            for sl in slots:
                acc = acc + recv[sl]
            comm[phase_idx + 1] = acc

        def finalize(b, comm):
            tot = comm[N_PHASES]
            recip = 1.0 / tot[:, HD:HD + Hq].astype(jnp.float32)
            mult = jnp.broadcast_to(
                recip[:, :, None], (Sq, Hq, Dh)
            ).reshape(Sq, HD)
            ctx_buf[b * Sq:(b + 1) * Sq, :] = (
                tot[:, :HD].astype(jnp.float32) * mult
            )

        partials(0, commA)
        pl.semaphore_wait(barrier_sem, len(all_peers))
        rsA = launch(commA, recvA, sendA_sems, recvA_sems, 0, phases_A[0])
        partials(1, commB)
        rsB = launch(commB, recvB, sendB_sems, recvB_sems, 0, phases_B[0])

        rdmas = list(rsA) + list(rsB)
        for ph in range(N_PHASES):
            finish(commA, recvA, ph, phases_A[ph], rsA)
            if ph + 1 < N_PHASES:
                rsA = launch(commA, recvA, sendA_sems, recvA_sems,
                             ph + 1, phases_A[ph + 1])
                rdmas += rsA
            else:
                finalize(0, commA)
            finish(commB, recvB, ph, phases_B[ph], rsB)
            if ph + 1 < N_PHASES:
                rsB = launch(commB, recvB, sendB_sems, recvB_sems,
                             ph + 1, phases_B[ph + 1])
                rdmas += rsB
        finalize(1, commB)
        for r in rdmas:
            r.wait_send()

        out = jnp.dot(
            ctx_buf[...], wo_ref[...], preferred_element_type=jnp.float32
        )
        out_ref[...] = out.reshape(B, Sq, Dm)

    return pl.pallas_call(
        body,
        out_shape=jax.ShapeDtypeStruct((B, Sq, Dm), jnp.float32),
        in_specs=[pl.BlockSpec(memory_space=pltpu.VMEM)] * 5,
        out_specs=pl.BlockSpec(memory_space=pltpu.VMEM),
        scratch_shapes=[
            pltpu.VMEM((N_PHASES + 1, Sq, W_COLS), jnp.bfloat16),
            pltpu.VMEM((N_PHASES + 1, Sq, W_COLS), jnp.bfloat16),
            pltpu.VMEM((N_SLOTS, Sq, W_COLS), jnp.bfloat16),
            pltpu.VMEM((N_SLOTS, Sq, W_COLS), jnp.bfloat16),
            pltpu.VMEM((B * Sq, HD), jnp.float32),
            pltpu.SemaphoreType.DMA((N_SLOTS,)),
            pltpu.SemaphoreType.DMA((N_SLOTS,)),
            pltpu.SemaphoreType.DMA((N_SLOTS,)),
            pltpu.SemaphoreType.DMA((N_SLOTS,)),
        ],
        compiler_params=pltpu.CompilerParams(collective_id=0),
    )(x, Wq, K_ext, V_ext, Wo)


# device time: 24537 ns/iter; 1.1818x vs baseline; 1.0014x over previous
import jax
import jax.numpy as jnp
from jax import lax
from jax.experimental import pallas as pl
from jax.experimental.pallas import tpu as pltpu

N_DEV = 32
N_PHASES = 3
N_SLOTS = 7
BLK = 64


def kernel(x, Wq, K_ext, V_ext, Wo):
    B, Sq, Dm = x.shape
    _, Skv_loc, Hq, Dh = K_ext.shape
    HD = Hq * Dh
    W_COLS = HD + Hq

    def body(x_ref, wq_ref, k_ref, v_ref, wo_ref, out_ref,
             commA, commB, recvA, recvB, ctx_buf,
             sendA_sems, recvA_sems, sendB_sems, recvB_sems):
        my = lax.axis_index("i")

        zz = my // 8
        p = my % 8
        yy = p // 2
        xx = (p % 2 + yy) % 2

        def snake(xv, yv):
            return 2 * yv + (xv + yv) % 2

        peers_x = [my ^ 1]
        peers_y = [8 * zz + snake(xx, (yy + d) % 4) for d in (1, 2, 3)]
        peers_z = [8 * ((zz + d) % 4) + p for d in (1, 2, 3)]

        PH_X = (peers_x, [0])
        PH_Y = (peers_y, [1, 2, 3])
        PH_Z = (peers_z, [4, 5, 6])
        phases_A = [PH_Y, PH_Z, PH_X]
        phases_B = [PH_Z, PH_Y, PH_X]

        barrier_sem = pltpu.get_barrier_semaphore()
        all_peers = peers_x + peers_y + peers_z
        for tgt in all_peers:
            pl.semaphore_signal(
                barrier_sem, inc=1,
                device_id=(tgt,),
                device_id_type=pl.DeviceIdType.MESH,
            )

        xf = x_ref[...].reshape(B * Sq, Dm)
        q2 = jnp.dot(xf, wq_ref[...], preferred_element_type=jnp.float32)
        k2 = k_ref[...].reshape(B, Skv_loc, HD)
        v2 = v_ref[...].reshape(B, Skv_loc, HD)

        qb = lax.broadcasted_iota(jnp.int32, (Sq, Skv_loc), 0) // BLK
        kb = my * (Skv_loc // BLK) + (
            lax.broadcasted_iota(jnp.int32, (Sq, Skv_loc), 1) // BLK
        )
        mask = (qb == kb) | (kb == 0) | ((qb + kb) % 3 == 0)

        def partials(b, comm):
            q_b = q2[b * Sq:(b + 1) * Sq, :]
            k_b = k2[b]
            v_b = v2[b]
            for h in range(Hq):
                cols = slice(h * Dh, (h + 1) * Dh)
                s = lax.dot_general(
                    q_b[:, cols], k_b[:, cols],
                    (((1,), (1,)), ((), ())),
                    preferred_element_type=jnp.float32,
                ) * 0.125
                w = jnp.where(mask, jnp.exp(s), 0.0)
                l_h = jnp.sum(w, axis=1)
                c_h = jnp.dot(
                    w, v_b[:, cols], preferred_element_type=jnp.float32
                )
                comm[0, :, cols] = c_h.astype(jnp.bfloat16)
                comm[0, :, HD + h:HD + h + 1] = (
                    l_h[:, None].astype(jnp.bfloat16)
                )

        def launch(comm, recv, ssems, rsems, phase_idx, phase):
            peers, slots = phase
            rs = []
            for tgt, sl in zip(peers, slots):
                r = pltpu.make_async_remote_copy(
                    src_ref=comm.at[phase_idx],
                    dst_ref=recv.at[sl],
                    send_sem=ssems.at[sl],
                    recv_sem=rsems.at[sl],
                    device_id=(tgt,),
                    device_id_type=pl.DeviceIdType.MESH,
                )
                r.start()
                rs.append(r)
            return rs

        def finish(comm, recv, phase_idx, phase, rs):
            _, slots = phase
            for r in rs:
                r.wait_recv()
            acc = comm[phase_idx]
            for sl in slots:
                acc = acc + recv[sl]
            comm[phase_idx + 1] = acc

        def finalize(b, comm):
            tot = comm[N_PHASES]
            recip = 1.0 / tot[:, HD:HD + Hq].astype(jnp.float32)
            mult = jnp.broadcast_to(
                recip[:, :, None], (Sq, Hq, Dh)
            ).reshape(Sq, HD)
            ctx_buf[b * Sq:(b + 1) * Sq, :] = (
                tot[:, :HD].astype(jnp.float32) * mult
            )

        partials(0, commA)
        pl.semaphore_wait(barrier_sem, len(all_peers))
        rsA = launch(commA, recvA, sendA_sems, recvA_sems, 0, phases_A[0])
        partials(1, commB)
        rsB = launch(commB, recvB, sendB_sems, recvB_sems, 0, phases_B[0])

        rdmas = list(rsA) + list(rsB)
        for ph in range(N_PHASES):
            finish(commA, recvA, ph, phases_A[ph], rsA)
            if ph + 1 < N_PHASES:
                rsA = launch(commA, recvA, sendA_sems, recvA_sems,
                             ph + 1, phases_A[ph + 1])
                rdmas += rsA
            else:
                finalize(0, commA)
            finish(commB, recvB, ph, phases_B[ph], rsB)
            if ph + 1 < N_PHASES:
                rsB = launch(commB, recvB, sendB_sems, recvB_sems,
                             ph + 1, phases_B[ph + 1])
                rdmas += rsB
        finalize(1, commB)
        for r in rdmas:
            r.wait_send()

        out = jnp.dot(
            ctx_buf[...], wo_ref[...], preferred_element_type=jnp.float32
        )
        out_ref[...] = out.reshape(B, Sq, Dm)

    return pl.pallas_call(
        body,
        out_shape=jax.ShapeDtypeStruct((B, Sq, Dm), jnp.float32),
        in_specs=[pl.BlockSpec(memory_space=pltpu.VMEM)] * 5,
        out_specs=pl.BlockSpec(memory_space=pltpu.VMEM),
        scratch_shapes=[
            pltpu.VMEM((N_PHASES + 1, Sq, W_COLS), jnp.bfloat16),
            pltpu.VMEM((N_PHASES + 1, Sq, W_COLS), jnp.bfloat16),
            pltpu.VMEM((N_SLOTS, Sq, W_COLS), jnp.bfloat16),
            pltpu.VMEM((N_SLOTS, Sq, W_COLS), jnp.bfloat16),
            pltpu.VMEM((B * Sq, HD), jnp.float32),
            pltpu.SemaphoreType.DMA((N_SLOTS,)),
            pltpu.SemaphoreType.DMA((N_SLOTS,)),
            pltpu.SemaphoreType.DMA((N_SLOTS,)),
            pltpu.SemaphoreType.DMA((N_SLOTS,)),
        ],
        compiler_params=pltpu.CompilerParams(collective_id=0),
    )(x, Wq, K_ext, V_ext, Wo)


# device time: 24374 ns/iter; 1.1897x vs baseline; 1.0067x over previous
import jax
import jax.numpy as jnp
from jax import lax
from jax.experimental import pallas as pl
from jax.experimental.pallas import tpu as pltpu

N_DEV = 32
N_PHASES = 3
N_SLOTS = 7
BLK = 64


def kernel(x, Wq, K_ext, V_ext, Wo):
    B, Sq, Dm = x.shape
    _, Skv_loc, Hq, Dh = K_ext.shape
    HD = Hq * Dh
    W_COLS = HD + Hq

    def body(x_ref, wq_ref, k_ref, v_ref, wo_ref, out_ref,
             commA, commB, recvA, recvB,
             sendA_sems, recvA_sems, sendB_sems, recvB_sems):
        my = lax.axis_index("i")

        zz = my // 8
        p = my % 8
        yy = p // 2
        xx = (p % 2 + yy) % 2

        def snake(xv, yv):
            return 2 * yv + (xv + yv) % 2

        peers_x = [my ^ 1]
        peers_y = [8 * zz + snake(xx, (yy + d) % 4) for d in (1, 2, 3)]
        peers_z = [8 * ((zz + d) % 4) + p for d in (1, 2, 3)]

        PH_X = (peers_x, [0])
        PH_Y = (peers_y, [1, 2, 3])
        PH_Z = (peers_z, [4, 5, 6])
        phases_A = [PH_Y, PH_Z, PH_X]
        phases_B = [PH_Z, PH_Y, PH_X]

        barrier_sem = pltpu.get_barrier_semaphore()
        all_peers = peers_x + peers_y + peers_z
        for tgt in all_peers:
            pl.semaphore_signal(
                barrier_sem, inc=1,
                device_id=(tgt,),
                device_id_type=pl.DeviceIdType.MESH,
            )

        xf = x_ref[...].reshape(B * Sq, Dm)
        q2 = jnp.dot(xf, wq_ref[...], preferred_element_type=jnp.float32)
        k2 = k_ref[...].reshape(B, Skv_loc, HD)
        v2 = v_ref[...].reshape(B, Skv_loc, HD)

        qb = lax.broadcasted_iota(jnp.int32, (Sq, Skv_loc), 0) // BLK
        kb = my * (Skv_loc // BLK) + (
            lax.broadcasted_iota(jnp.int32, (Sq, Skv_loc), 1) // BLK
        )
        mask = (qb == kb) | (kb == 0) | ((qb + kb) % 3 == 0)

        def partials(b, comm):
            q_b = q2[b * Sq:(b + 1) * Sq, :]
            k_b = k2[b]
            v_b = v2[b]
            for h in range(Hq):
                cols = slice(h * Dh, (h + 1) * Dh)
                s = lax.dot_general(
                    q_b[:, cols], k_b[:, cols],
                    (((1,), (1,)), ((), ())),
                    preferred_element_type=jnp.float32,
                ) * 0.125
                w = jnp.where(mask, jnp.exp(s), 0.0)
                l_h = jnp.sum(w, axis=1)
                c_h = jnp.dot(
                    w, v_b[:, cols], preferred_element_type=jnp.float32
                )
                comm[0, :, cols] = c_h.astype(jnp.bfloat16)
                comm[0, :, HD + h:HD + h + 1] = (
                    l_h[:, None].astype(jnp.bfloat16)
                )

        def launch(comm, recv, ssems, rsems, phase_idx, phase):
            peers, slots = phase
            rs = []
            for tgt, sl in zip(peers, slots):
                r = pltpu.make_async_remote_copy(
                    src_ref=comm.at[phase_idx],
                    dst_ref=recv.at[sl],
                    send_sem=ssems.at[sl],
                    recv_sem=rsems.at[sl],
                    device_id=(tgt,),
                    device_id_type=pl.DeviceIdType.MESH,
                )
                r.start()
                rs.append(r)
            return rs

        def finish(comm, recv, phase_idx, phase, rs):
            _, slots = phase
            for r in rs:
                r.wait_recv()
            acc = comm[phase_idx]
            for sl in slots:
                acc = acc + recv[sl]
            comm[phase_idx + 1] = acc

        def finalize(b, comm):
            tot = comm[N_PHASES]
            recip = 1.0 / tot[:, HD:HD + Hq].astype(jnp.float32)
            mult = jnp.broadcast_to(
                recip[:, :, None], (Sq, Hq, Dh)
            ).reshape(Sq, HD)
            ctx = tot[:, :HD].astype(jnp.float32) * mult
            out_ref[b] = jnp.dot(
                ctx, wo_ref[...], preferred_element_type=jnp.float32
            )

        partials(0, commA)
        pl.semaphore_wait(barrier_sem, len(all_peers))
        rsA = launch(commA, recvA, sendA_sems, recvA_sems, 0, phases_A[0])
        partials(1, commB)
        rsB = launch(commB, recvB, sendB_sems, recvB_sems, 0, phases_B[0])

        rdmas = list(rsA) + list(rsB)
        for ph in range(N_PHASES):
            finish(commA, recvA, ph, phases_A[ph], rsA)
            if ph + 1 < N_PHASES:
                rsA = launch(commA, recvA, sendA_sems, recvA_sems,
                             ph + 1, phases_A[ph + 1])
                rdmas += rsA
            else:
                finalize(0, commA)
            finish(commB, recvB, ph, phases_B[ph], rsB)
            if ph + 1 < N_PHASES:
                rsB = launch(commB, recvB, sendB_sems, recvB_sems,
                             ph + 1, phases_B[ph + 1])
                rdmas += rsB
        finalize(1, commB)
        for r in rdmas:
            r.wait_send()

    return pl.pallas_call(
        body,
        out_shape=jax.ShapeDtypeStruct((B, Sq, Dm), jnp.float32),
        in_specs=[pl.BlockSpec(memory_space=pltpu.VMEM)] * 5,
        out_specs=pl.BlockSpec(memory_space=pltpu.VMEM),
        scratch_shapes=[
            pltpu.VMEM((N_PHASES + 1, Sq, W_COLS), jnp.bfloat16),
            pltpu.VMEM((N_PHASES + 1, Sq, W_COLS), jnp.bfloat16),
            pltpu.VMEM((N_SLOTS, Sq, W_COLS), jnp.bfloat16),
            pltpu.VMEM((N_SLOTS, Sq, W_COLS), jnp.bfloat16),
            pltpu.SemaphoreType.DMA((N_SLOTS,)),
            pltpu.SemaphoreType.DMA((N_SLOTS,)),
            pltpu.SemaphoreType.DMA((N_SLOTS,)),
            pltpu.SemaphoreType.DMA((N_SLOTS,)),
        ],
        compiler_params=pltpu.CompilerParams(collective_id=0),
    )(x, Wq, K_ext, V_ext, Wo)


# device time: 24358 ns/iter; 1.1905x vs baseline; 1.0007x over previous
import jax
import jax.numpy as jnp
from jax import lax
from jax.experimental import pallas as pl
from jax.experimental.pallas import tpu as pltpu

N_DEV = 32
N_PHASES = 3
N_SLOTS = 7
BLK = 64


def kernel(x, Wq, K_ext, V_ext, Wo):
    B, Sq, Dm = x.shape
    _, Skv_loc, Hq, Dh = K_ext.shape
    HD = Hq * Dh
    W_COLS = HD + Hq

    def body(x_ref, wq_ref, k_ref, v_ref, wo_ref, out_ref,
             commA, commB, recvA, recvB,
             sendA_sems, recvA_sems, sendB_sems, recvB_sems):
        my = lax.axis_index("i")

        zz = my // 8
        p = my % 8
        yy = p // 2
        xx = (p % 2 + yy) % 2

        def snake(xv, yv):
            return 2 * yv + (xv + yv) % 2

        peers_x = [my ^ 1]
        peers_y = [8 * zz + snake(xx, (yy + d) % 4) for d in (1, 2, 3)]
        peers_z = [8 * ((zz + d) % 4) + p for d in (1, 2, 3)]

        PH_X = (peers_x, [0])
        PH_Y = (peers_y, [1, 2, 3])
        PH_Z = (peers_z, [4, 5, 6])
        phases_A = [PH_Y, PH_Z, PH_X]
        phases_B = [PH_Z, PH_Y, PH_X]

        barrier_sem = pltpu.get_barrier_semaphore()
        all_peers = peers_x + peers_y + peers_z
        for tgt in all_peers:
            pl.semaphore_signal(
                barrier_sem, inc=1,
                device_id=(tgt,),
                device_id_type=pl.DeviceIdType.MESH,
            )

        xf = x_ref[...].reshape(B * Sq, Dm)
        q2 = jnp.dot(xf, wq_ref[...], preferred_element_type=jnp.float32)
        k2 = k_ref[...].reshape(B, Skv_loc, HD)
        v2 = v_ref[...].reshape(B, Skv_loc, HD)

        qb = lax.broadcasted_iota(jnp.int32, (Sq, Skv_loc), 0) // BLK
        kb = my * (Skv_loc // BLK) + (
            lax.broadcasted_iota(jnp.int32, (Sq, Skv_loc), 1) // BLK
        )
        mask = (qb == kb) | (kb == 0) | ((qb + kb) % 3 == 0)

        def partials(b, comm):
            q_b = q2[b * Sq:(b + 1) * Sq, :]
            k_b = k2[b]
            v_b = v2[b]
            for h in range(Hq):
                cols = slice(h * Dh, (h + 1) * Dh)
                s = lax.dot_general(
                    q_b[:, cols], k_b[:, cols],
                    (((1,), (1,)), ((), ())),
                    preferred_element_type=jnp.float32,
                ) * 0.125
                w = jnp.where(mask, jnp.exp(s), 0.0)
                l_h = jnp.sum(w, axis=1)
                c_h = jnp.dot(
                    w, v_b[:, cols], preferred_element_type=jnp.float32
                )
                comm[0, :, cols] = c_h.astype(jnp.bfloat16)
                comm[0, :, HD + h:HD + h + 1] = (
                    l_h[:, None].astype(jnp.bfloat16)
                )

        def launch(comm, recv, ssems, rsems, phase_idx, phase):
            peers, slots = phase
            rs = []
            for tgt, sl in zip(peers, slots):
                r = pltpu.make_async_remote_copy(
                    src_ref=comm.at[phase_idx],
                    dst_ref=recv.at[sl],
                    send_sem=ssems.at[sl],
                    recv_sem=rsems.at[sl],
                    device_id=(tgt,),
                    device_id_type=pl.DeviceIdType.MESH,
                )
                r.start()
                rs.append(r)
            return rs

        def finish(comm, recv, phase_idx, phase, rs):
            _, slots = phase
            acc = comm[phase_idx]
            for r, sl in zip(rs, slots):
                r.wait_recv()
                acc = acc + recv[sl]
            comm[phase_idx + 1] = acc

        def finalize(b, comm):
            tot = comm[N_PHASES]
            recip = 1.0 / tot[:, HD:HD + Hq].astype(jnp.float32)
            mult = jnp.broadcast_to(
                recip[:, :, None], (Sq, Hq, Dh)
            ).reshape(Sq, HD)
            ctx = tot[:, :HD].astype(jnp.float32) * mult
            out_ref[b] = jnp.dot(
                ctx, wo_ref[...], preferred_element_type=jnp.float32
            )

        partials(0, commA)
        pl.semaphore_wait(barrier_sem, len(all_peers))
        rsA = launch(commA, recvA, sendA_sems, recvA_sems, 0, phases_A[0])
        partials(1, commB)
        rsB = launch(commB, recvB, sendB_sems, recvB_sems, 0, phases_B[0])

        rdmas = list(rsA) + list(rsB)
        for ph in range(N_PHASES):
            finish(commA, recvA, ph, phases_A[ph], rsA)
            if ph + 1 < N_PHASES:
                rsA = launch(commA, recvA, sendA_sems, recvA_sems,
                             ph + 1, phases_A[ph + 1])
                rdmas += rsA
            else:
                finalize(0, commA)
            finish(commB, recvB, ph, phases_B[ph], rsB)
            if ph + 1 < N_PHASES:
                rsB = launch(commB, recvB, sendB_sems, recvB_sems,
                             ph + 1, phases_B[ph + 1])
                rdmas += rsB
        finalize(1, commB)
        for r in rdmas:
            r.wait_send()

    return pl.pallas_call(
        body,
        out_shape=jax.ShapeDtypeStruct((B, Sq, Dm), jnp.float32),
        in_specs=[pl.BlockSpec(memory_space=pltpu.VMEM)] * 5,
        out_specs=pl.BlockSpec(memory_space=pltpu.VMEM),
        scratch_shapes=[
            pltpu.VMEM((N_PHASES + 1, Sq, W_COLS), jnp.bfloat16),
            pltpu.VMEM((N_PHASES + 1, Sq, W_COLS), jnp.bfloat16),
            pltpu.VMEM((N_SLOTS, Sq, W_COLS), jnp.bfloat16),
            pltpu.VMEM((N_SLOTS, Sq, W_COLS), jnp.bfloat16),
            pltpu.SemaphoreType.DMA((N_SLOTS,)),
            pltpu.SemaphoreType.DMA((N_SLOTS,)),
            pltpu.SemaphoreType.DMA((N_SLOTS,)),
            pltpu.SemaphoreType.DMA((N_SLOTS,)),
        ],
        compiler_params=pltpu.CompilerParams(collective_id=0),
    )(x, Wq, K_ext, V_ext, Wo)
